# baseline (speedup 1.0000x reference)
import functools

import jax
import jax.numpy as jnp
from jax import lax
from jax.experimental import pallas as pl
from jax.experimental.pallas import tpu as pltpu
from jax.experimental.pallas import tpu_sc as plsc

B, C, H, W = 16, 256, 32, 32
N = B * H * W
K = 1024
T = 4096
G = N // T
BETA = 0.25

NC, NS = 2, 16
NW = NC * NS
BPW = N // NW
CH = 128


def _vq_kernel(x_ref, cn_ref, cb_ref, ki_ref, idx_ref, loss_ref):
    g = pl.program_id(0)
    x = x_ref[...]
    cnorm = cn_ref[...]
    cb = cb_ref[...]
    kiota = ki_ref[...]

    xnorm = jnp.sum(x * x, axis=1, keepdims=True)
    xm2 = -2.0 * x
    xcm2 = lax.dot_general(xm2, cb, (((1,), (1,)), ((), ())),
                           preferred_element_type=jnp.float32)
    scores = (xnorm + cnorm) + xcm2

    minv = jnp.min(scores, axis=1, keepdims=True)
    idx_f = jnp.min(jnp.where(scores == minv, kiota, float(K)),
                    axis=1, keepdims=True)
    idx_ref[...] = jnp.transpose(idx_f.astype(jnp.int32), (1, 0))

    part = jnp.sum(minv).reshape(1, 1)

    @pl.when(g == 0)
    def _init():
        loss_ref[...] = part

    @pl.when(g != 0)
    def _acc():
        loss_ref[...] += part


def _sc_gather_kernel(table_hbm, idx_hbm, out_hbm, idx_v, rows_v, sem):
    wid = lax.axis_index("s") * NC + lax.axis_index("c")
    base = wid * BPW
    for j in range(BPW // CH):
        off = base + j * CH
        pltpu.sync_copy(idx_hbm.at[pl.ds(off, CH)], idx_v)
        pltpu.async_copy(table_hbm.at[idx_v], rows_v, sem).wait()
        pltpu.sync_copy(rows_v, out_hbm.at[pl.ds(off, CH)])


@jax.jit
def kernel(z, embedding_weight):
    x_flat = jnp.transpose(z, (0, 2, 3, 1)).reshape(N, C)
    kiota = lax.broadcasted_iota(jnp.float32, (1, K), 1)
    cnorm = jnp.sum(embedding_weight ** 2, axis=1).reshape(1, K)

    idx_row, loss_acc = pl.pallas_call(
        _vq_kernel,
        grid=(G,),
        in_specs=[
            pl.BlockSpec((T, C), lambda g: (g, 0)),
            pl.BlockSpec((1, K), lambda g: (0, 0)),
            pl.BlockSpec((K, C), lambda g: (0, 0)),
            pl.BlockSpec((1, K), lambda g: (0, 0)),
        ],
        out_specs=[
            pl.BlockSpec((1, T), lambda g: (0, g)),
            pl.BlockSpec((1, 1), lambda g: (0, 0)),
        ],
        out_shape=[
            jax.ShapeDtypeStruct((1, N), jnp.int32),
            jax.ShapeDtypeStruct((1, 1), jnp.float32),
        ],
    )(x_flat, cnorm, embedding_weight, kiota)

    idx_flat = idx_row.reshape(N)
    sc_gather = functools.partial(
        pl.kernel,
        mesh=plsc.VectorSubcoreMesh(core_axis_name="c", subcore_axis_name="s"),
        out_type=jax.ShapeDtypeStruct((N, C), jnp.float32),
        scratch_types=[
            pltpu.VMEM((CH,), jnp.int32),
            pltpu.VMEM((CH, C), jnp.float32),
            pltpu.SemaphoreType.DMA,
        ],
    )(_sc_gather_kernel)
    zq_flat = sc_gather(embedding_weight, idx_flat)

    z_q_out = zq_flat.reshape(B, H, W, C).transpose(0, 3, 1, 2)
    indices_out = idx_row.reshape(B, 1, H, W)
    codebook_loss = (1.0 + BETA) * loss_acc[0, 0] / (N * C)
    return (z_q_out, codebook_loss, indices_out)


# speedup vs baseline: 1.3870x; 1.3870x over previous
import jax
import jax.numpy as jnp
from jax import lax
from jax.experimental import pallas as pl
from jax.experimental.pallas import tpu as pltpu

B, C, H, W = 16, 256, 32, 32
N = B * H * W
K = 1024
T = 4096
G = N // T
BETA = 0.25


def _vq_kernel(x_ref, cn_ref, cb_ref, ki_ref, zq_ref, idx_ref,
               loss_ref, oh_ref):
    g = pl.program_id(0)

    @pl.when(g > 0)
    def _zq_prev():
        zq_ref[...] = lax.dot_general(
            oh_ref[...], cb_ref[...], (((1,), (0,)), ((), ())),
            preferred_element_type=jnp.float32)

    @pl.when(g < G)
    def _argmin_cur():
        x = x_ref[...]
        cnorm = cn_ref[...]
        cb = cb_ref[...]
        kiota = ki_ref[...]

        xnorm = jnp.sum(x * x, axis=1, keepdims=True)
        xm2 = -2.0 * x
        xcm2 = lax.dot_general(xm2, cb, (((1,), (1,)), ((), ())),
                               preferred_element_type=jnp.float32)
        scores = (xnorm + cnorm) + xcm2

        minv = jnp.min(scores, axis=1, keepdims=True)
        idx_f = jnp.min(jnp.where(scores == minv, kiota, float(K)),
                        axis=1, keepdims=True)
        idx_ref[...] = jnp.transpose(idx_f.astype(jnp.int32), (1, 0))
        oh_ref[...] = (kiota == idx_f).astype(jnp.bfloat16)

        part = jnp.sum(minv).reshape(1, 1)

        @pl.when(g == 0)
        def _init():
            loss_ref[...] = part

        @pl.when(g != 0)
        def _acc():
            loss_ref[...] += part


@jax.jit
def kernel(z, embedding_weight):
    x_flat = jnp.transpose(z, (0, 2, 3, 1)).reshape(N, C)
    kiota = lax.broadcasted_iota(jnp.float32, (1, K), 1)
    cnorm = jnp.sum(embedding_weight ** 2, axis=1).reshape(1, K)

    zq_flat, idx_col, loss_acc = pl.pallas_call(
        _vq_kernel,
        grid=(G + 1,),
        in_specs=[
            pl.BlockSpec((T, C), lambda g: (jnp.minimum(g, G - 1), 0)),
            pl.BlockSpec((1, K), lambda g: (0, 0)),
            pl.BlockSpec((K, C), lambda g: (0, 0)),
            pl.BlockSpec((1, K), lambda g: (0, 0)),
        ],
        out_specs=[
            pl.BlockSpec((T, C), lambda g: (jnp.maximum(g - 1, 0), 0)),
            pl.BlockSpec((1, T), lambda g: (0, jnp.minimum(g, G - 1))),
            pl.BlockSpec((1, 1), lambda g: (0, 0)),
        ],
        scratch_shapes=[pltpu.VMEM((T, K), jnp.bfloat16)],
        out_shape=[
            jax.ShapeDtypeStruct((N, C), jnp.float32),
            jax.ShapeDtypeStruct((1, N), jnp.int32),
            jax.ShapeDtypeStruct((1, 1), jnp.float32),
        ],
    )(x_flat, cnorm, embedding_weight, kiota)

    z_q_out = zq_flat.reshape(B, H, W, C).transpose(0, 3, 1, 2)
    indices_out = idx_col.reshape(B, 1, H, W)
    codebook_loss = (1.0 + BETA) * loss_acc[0, 0] / (N * C)
    return (z_q_out, codebook_loss, indices_out)


# speedup vs baseline: 1.4740x; 1.0627x over previous
import jax
import jax.numpy as jnp
from jax import lax
from jax.experimental import pallas as pl
from jax.experimental.pallas import tpu as pltpu

B, C, H, W = 16, 256, 32, 32
N = B * H * W
K = 1024
T = 4096
G = N // T
BETA = 0.25


def _vq_kernel(x_ref, cb_ref, zq_ref, idx_ref, loss_ref, oh_ref, cn_ref,
               ki_ref):
    g = pl.program_id(0)

    @pl.when(g == 0)
    def _setup():
        cbv = cb_ref[...]
        cn_col = jnp.sum(cbv * cbv, axis=1, keepdims=True)
        cn_ref[...] = jnp.transpose(cn_col, (1, 0))
        kiota_col = lax.broadcasted_iota(jnp.int32, (K, 1), 0)
        ki_ref[...] = jnp.transpose(kiota_col.astype(jnp.float32), (1, 0))

    @pl.when(g > 0)
    def _zq_prev():
        zq_ref[...] = lax.dot_general(
            oh_ref[...], cb_ref[...], (((1,), (0,)), ((), ())),
            preferred_element_type=jnp.float32)

    @pl.when(g < G)
    def _argmin_cur():
        x = x_ref[...]
        cnorm = cn_ref[...]
        cb = cb_ref[...]
        kiota = ki_ref[...]

        xnorm = jnp.sum(x * x, axis=1, keepdims=True)
        xm2 = -2.0 * x
        xcm2 = lax.dot_general(xm2, cb, (((1,), (1,)), ((), ())),
                               preferred_element_type=jnp.float32)
        scores = (xnorm + cnorm) + xcm2

        minv = jnp.min(scores, axis=1, keepdims=True)
        idx_f = jnp.min(jnp.where(scores == minv, kiota, float(K)),
                        axis=1, keepdims=True)
        idx_ref[...] = jnp.transpose(idx_f.astype(jnp.int32), (1, 0))
        oh_ref[...] = (kiota == idx_f).astype(jnp.float32)

        part = jnp.sum(minv).reshape(1, 1)

        @pl.when(g == 0)
        def _init():
            loss_ref[...] = part

        @pl.when(g != 0)
        def _acc():
            loss_ref[...] += part


@jax.jit
def kernel(z, embedding_weight):
    x_flat = jnp.transpose(z, (0, 2, 3, 1)).reshape(N, C)

    zq_flat, idx_col, loss_acc = pl.pallas_call(
        _vq_kernel,
        grid=(G + 1,),
        in_specs=[
            pl.BlockSpec((T, C), lambda g: (jnp.minimum(g, G - 1), 0)),
            pl.BlockSpec((K, C), lambda g: (0, 0)),
        ],
        out_specs=[
            pl.BlockSpec((T, C), lambda g: (jnp.maximum(g - 1, 0), 0)),
            pl.BlockSpec((1, T), lambda g: (0, jnp.minimum(g, G - 1))),
            pl.BlockSpec((1, 1), lambda g: (0, 0)),
        ],
        scratch_shapes=[pltpu.VMEM((T, K), jnp.float32),
                        pltpu.VMEM((1, K), jnp.float32),
                        pltpu.VMEM((1, K), jnp.float32)],
        out_shape=[
            jax.ShapeDtypeStruct((N, C), jnp.float32),
            jax.ShapeDtypeStruct((1, N), jnp.int32),
            jax.ShapeDtypeStruct((1, 1), jnp.float32),
        ],
    )(x_flat, embedding_weight)

    z_q_out = zq_flat.reshape(B, H, W, C).transpose(0, 3, 1, 2)
    indices_out = idx_col.reshape(B, 1, H, W)
    codebook_loss = (1.0 + BETA) * loss_acc[0, 0] / (N * C)
    return (z_q_out, codebook_loss, indices_out)
